# baseline (speedup 1.0000x reference)
import jax
import jax.numpy as jnp
from jax.experimental import pallas as pl
from jax.experimental.pallas import tpu as pltpu

NUM_CLASSES = 80
ALPHA = 0.25
TOPK = 10
BIGI = 10 ** 9
REMOVED = 3.0e38


def _dot(a, b):
    return jax.lax.dot_general(a, b, (((1,), (0,)), ((), ())),
                               preferred_element_type=jnp.float32)


def _ota_kernel(pred_cls_ref, pbox_ref, anc_ref, mask_ref, glab_ref,
                gbox_ref, cls_ref, reg_ref, acc_ref):
    M = pbox_ref.shape[1]
    G = gbox_ref.shape[1]
    B = pl.num_programs(0)
    b = pl.program_id(0)

    x_cls = pred_cls_ref[0]
    inv_mask = 1.0 - mask_ref[0]
    glab = glab_ref[0]
    gbox = gbox_ref[0]

    iota_m = jax.lax.broadcasted_iota(jnp.int32, (1, M), 1)
    iota_g = jax.lax.broadcasted_iota(jnp.int32, (G, 1), 0)
    ones_m = jnp.ones((M, 1), jnp.float32)

    eye4 = (jax.lax.broadcasted_iota(jnp.int32, (4, 4), 0)
            == jax.lax.broadcasted_iota(jnp.int32, (4, 4), 1)).astype(jnp.float32)
    pbt = jax.lax.dot_general(eye4, pbox_ref[0], (((1,), (1,)), ((), ())),
                              preferred_element_type=jnp.float32)
    anc = jax.lax.dot_general(eye4[0:2], anc_ref[...], (((1,), (1,)), ((), ())),
                              preferred_element_type=jnp.float32)

    ax = anc[0:1, :]
    ay = anc[1:2, :]
    px1, py1, px2, py2 = (pbt[0:1, :], pbt[1:2, :], pbt[2:3, :], pbt[3:4, :])
    gx1, gy1, gx2, gy2 = (gbox[:, 0:1], gbox[:, 1:2], gbox[:, 2:3], gbox[:, 3:4])

    d = jnp.minimum(jnp.minimum(ax - gx1, ay - gy1),
                    jnp.minimum(gx2 - ax, gy2 - ay))
    is_in_gt = d > 0.0
    valid_any = jnp.max(jnp.where(is_in_gt, 1.0, 0.0), axis=0, keepdims=True)

    area_g = jnp.clip(gx2 - gx1, 0.0) * jnp.clip(gy2 - gy1, 0.0)
    area_p = jnp.clip(px2 - px1, 0.0) * jnp.clip(py2 - py1, 0.0)
    iw = jnp.clip(jnp.minimum(gx2, px2) - jnp.maximum(gx1, px1), 0.0)
    ih = jnp.clip(jnp.minimum(gy2, py2) - jnp.maximum(gy1, py1), 0.0)
    inter = iw * ih
    union = area_g + area_p - inter
    ious = inter / jnp.clip(union, 1e-8)

    iota_c = jax.lax.broadcasted_iota(jnp.int32, (G, NUM_CLASSES), 1)
    onehot_lab = (iota_c == glab).astype(jnp.float32)
    ll = jax.lax.dot_general(
        onehot_lab, x_cls, (((1,), (1,)), ((), ())),
        preferred_element_type=jnp.float32)

    la = jnp.abs(ll)
    le = jnp.exp(-la)
    lL = jnp.log(1.0 + le)
    lmax0 = jnp.maximum(ll, 0.0)
    lnum = jnp.where(ll >= 0.0, 1.0, le)
    lp = lnum / (1.0 + le)

    sf = (ious - lp) * (ious - lp)
    ce = lmax0 - ll * ious + lL
    cost = ce * sf - 3.0 * jnp.log(ious + 1e-8)
    cost = jnp.where(is_in_gt & (valid_any > 0.0), cost, cost + 1e8)

    work = ious
    s10 = jnp.zeros((G, 1), jnp.float32)
    cum = jnp.zeros((G, 1), jnp.float32)
    for _ in range(TOPK):
        vmax = jnp.max(work, axis=1, keepdims=True)
        eq = work == vmax
        cnt = _dot(jnp.where(eq, 1.0, 0.0), ones_m)
        s10 = s10 + vmax * jnp.clip(float(TOPK) - cum, 0.0, cnt)
        cum = cum + cnt
        work = jnp.where(eq, -1.0, work)
    ks = jnp.clip(jnp.floor(s10), 1.0, float(M))

    work = cost
    match = jnp.zeros((G, M), jnp.float32)
    for i in range(TOPK):
        vmin = jnp.min(work, axis=1, keepdims=True)
        imin = jnp.min(jnp.where(work == vmin, iota_m, BIGI),
                       axis=1, keepdims=True)
        sel = iota_m == imin
        kmask = jnp.where(float(i) < ks, 1.0, 0.0)
        match = match + jnp.where(sel, kmask, 0.0)
        work = jnp.where(sel, REMOVED, work)

    ones_g = jnp.ones((1, G), jnp.float32)
    n_match = _dot(ones_g, match)
    cmin = jnp.min(cost, axis=0, keepdims=True)
    gmin = jnp.min(jnp.where(cost == cmin, iota_g, BIGI),
                   axis=0, keepdims=True)
    keep = (iota_g == gmin).astype(jnp.float32)
    match = jnp.where(n_match > 1.0, keep, match)
    fg = _dot(ones_g, match)

    x = x_cls
    a = jnp.abs(x)
    e = jnp.exp(-a)
    L = jnp.log(1.0 + e)
    num = jnp.where(x >= 0.0, 1.0, e)
    p = num / (1.0 + e)
    loss_bg = (1.0 - ALPHA) * (jnp.maximum(x, 0.0) + L) * p * p
    s_bg = _dot(loss_bg, jnp.ones((NUM_CLASSES, 1), jnp.float32))
    bg_sum = _dot(inv_mask, s_bg)

    q = 1.0 - lp
    l1 = ALPHA * (lmax0 - ll + lL) * q * q
    l0 = (1.0 - ALPHA) * (lmax0 + lL) * lp * lp
    corr = _dot(ones_g, match * (l1 - l0))
    cls_sum = bg_sum + jnp.sum(inv_mask * corr, axis=1, keepdims=True)

    bt = jax.lax.dot_general(gbox, match, (((0,), (0,)), ((), ())),
                             preferred_element_type=jnp.float32)
    tx1, ty1, tx2, ty2 = (bt[0:1, :], bt[1:2, :], bt[2:3, :], bt[3:4, :])
    a2 = jnp.clip(tx2 - tx1, 0.0) * jnp.clip(ty2 - ty1, 0.0)
    iw = jnp.clip(jnp.minimum(px2, tx2) - jnp.maximum(px1, tx1), 0.0)
    ih = jnp.clip(jnp.minimum(py2, ty2) - jnp.maximum(py1, ty1), 0.0)
    inter = iw * ih
    union = area_p + a2 - inter
    iou = inter / jnp.clip(union, 1e-8)
    cw = jnp.clip(jnp.maximum(px2, tx2) - jnp.minimum(px1, tx1), 0.0)
    ch = jnp.clip(jnp.maximum(py2, ty2) - jnp.minimum(py1, ty1), 0.0)
    carea = cw * ch
    gi = iou - (carea - union) / jnp.clip(carea, 1e-8)
    reg_sum = jnp.sum(fg * (1.0 - gi), axis=1, keepdims=True)

    npos = jnp.sum(fg, axis=1, keepdims=True)

    @pl.when(b == 0)
    def _init():
        acc_ref[...] = jnp.zeros_like(acc_ref)

    acc_ref[0:1, :] += cls_sum
    acc_ref[1:2, :] += reg_sum
    acc_ref[2:3, :] += npos

    @pl.when(b == B - 1)
    def _fin():
        num_fgs = jnp.maximum(acc_ref[2:3, :], 1.0)
        cls_ref[...] = acc_ref[0:1, :] / num_fgs
        reg_ref[...] = acc_ref[1:2, :] / num_fgs


@jax.jit
def kernel(pred_cls, pred_box, anchors, mask, gt_labels, gt_bboxes):
    B, M, C = pred_cls.shape
    G = gt_bboxes.shape[1]

    mask_f = mask.astype(jnp.float32).reshape(B, 1, M)
    glab = gt_labels.astype(jnp.int32).reshape(B, G, 1)

    out_sd = jax.ShapeDtypeStruct((1, 1), jnp.float32)
    cls_s, reg_s = pl.pallas_call(
        _ota_kernel,
        grid=(B,),
        in_specs=[
            pl.BlockSpec((1, M, C), lambda b: (b, 0, 0)),
            pl.BlockSpec((1, M, 4), lambda b: (b, 0, 0)),
            pl.BlockSpec((M, 4), lambda b: (0, 0)),
            pl.BlockSpec((1, 1, M), lambda b: (b, 0, 0)),
            pl.BlockSpec((1, G, 1), lambda b: (b, 0, 0)),
            pl.BlockSpec((1, G, 4), lambda b: (b, 0, 0)),
        ],
        out_specs=[
            pl.BlockSpec((1, 1), lambda b: (0, 0)),
            pl.BlockSpec((1, 1), lambda b: (0, 0)),
        ],
        out_shape=[out_sd, out_sd],
        scratch_shapes=[pltpu.VMEM((3, 1), jnp.float32)],
    )(pred_cls, pred_box, anchors, mask_f, glab, gt_bboxes)

    return cls_s[0, 0], reg_s[0, 0]


# speedup vs baseline: 1.1865x; 1.1865x over previous
import jax
import jax.numpy as jnp
from jax.experimental import pallas as pl
from jax.experimental.pallas import tpu as pltpu

NUM_CLASSES = 80
ALPHA = 0.25
TOPK = 10
BIGI = 10 ** 9
REMOVED = 3.0e38


def _dot(a, b):
    return jax.lax.dot_general(a, b, (((1,), (0,)), ((), ())),
                               preferred_element_type=jnp.float32)


def _ota_kernel(pred_cls_ref, pbox_ref, anc_ref, mask_ref, glab_ref,
                gbox_ref, cls_ref, reg_ref, acc_ref):
    M = pbox_ref.shape[2]
    G = gbox_ref.shape[1]
    B = pl.num_programs(0)
    b = pl.program_id(0)

    x_cls = pred_cls_ref[0]
    inv_mask = 1.0 - mask_ref[0]
    glab = glab_ref[0]
    gbox = gbox_ref[0]

    iota_m = jax.lax.broadcasted_iota(jnp.int32, (1, M), 1)
    iota_g = jax.lax.broadcasted_iota(jnp.int32, (G, 1), 0)
    ones_m = jnp.ones((M, 1), jnp.float32)

    pbt = pbox_ref[0]
    anc = anc_ref[...]

    ax = anc[0:1, :]
    ay = anc[1:2, :]
    px1, py1, px2, py2 = (pbt[0:1, :], pbt[1:2, :], pbt[2:3, :], pbt[3:4, :])
    gx1, gy1, gx2, gy2 = (gbox[:, 0:1], gbox[:, 1:2], gbox[:, 2:3], gbox[:, 3:4])

    d = jnp.minimum(jnp.minimum(ax - gx1, ay - gy1),
                    jnp.minimum(gx2 - ax, gy2 - ay))
    is_in_gt = d > 0.0
    valid_any = jnp.max(jnp.where(is_in_gt, 1.0, 0.0), axis=0, keepdims=True)

    area_g = jnp.clip(gx2 - gx1, 0.0) * jnp.clip(gy2 - gy1, 0.0)
    area_p = jnp.clip(px2 - px1, 0.0) * jnp.clip(py2 - py1, 0.0)
    iw = jnp.clip(jnp.minimum(gx2, px2) - jnp.maximum(gx1, px1), 0.0)
    ih = jnp.clip(jnp.minimum(gy2, py2) - jnp.maximum(gy1, py1), 0.0)
    inter = iw * ih
    union = area_g + area_p - inter
    ious = inter / jnp.clip(union, 1e-8)

    iota_c = jax.lax.broadcasted_iota(jnp.int32, (G, NUM_CLASSES), 1)
    onehot_lab = (iota_c == glab).astype(jnp.float32)
    ll = jax.lax.dot_general(
        onehot_lab, x_cls, (((1,), (1,)), ((), ())),
        preferred_element_type=jnp.float32)

    la = jnp.abs(ll)
    le = jnp.exp(-la)
    lL = jnp.log(1.0 + le)
    lmax0 = jnp.maximum(ll, 0.0)
    lnum = jnp.where(ll >= 0.0, 1.0, le)
    lp = lnum / (1.0 + le)

    sf = (ious - lp) * (ious - lp)
    ce = lmax0 - ll * ious + lL
    cost = ce * sf - 3.0 * jnp.log(ious + 1e-8)
    cost = jnp.where(is_in_gt & (valid_any > 0.0), cost, cost + 1e8)

    work = ious
    s10 = jnp.zeros((G, 1), jnp.float32)
    cum = jnp.zeros((G, 1), jnp.float32)
    for _ in range(TOPK):
        vmax = jnp.max(work, axis=1, keepdims=True)
        eq = work == vmax
        cnt = _dot(jnp.where(eq, 1.0, 0.0), ones_m)
        s10 = s10 + vmax * jnp.clip(float(TOPK) - cum, 0.0, cnt)
        cum = cum + cnt
        work = jnp.where(eq, -1.0, work)
    ks = jnp.clip(jnp.floor(s10), 1.0, float(M))

    work = cost
    match = jnp.zeros((G, M), jnp.float32)
    for i in range(TOPK):
        vmin = jnp.min(work, axis=1, keepdims=True)
        imin = jnp.min(jnp.where(work == vmin, iota_m, BIGI),
                       axis=1, keepdims=True)
        sel = iota_m == imin
        kmask = jnp.where(float(i) < ks, 1.0, 0.0)
        match = match + jnp.where(sel, kmask, 0.0)
        work = jnp.where(sel, REMOVED, work)

    ones_g = jnp.ones((1, G), jnp.float32)
    n_match = _dot(ones_g, match)
    cmin = jnp.min(cost, axis=0, keepdims=True)
    gmin = jnp.min(jnp.where(cost == cmin, iota_g, BIGI),
                   axis=0, keepdims=True)
    keep = (iota_g == gmin).astype(jnp.float32)
    match = jnp.where(n_match > 1.0, keep, match)
    fg = _dot(ones_g, match)

    x = x_cls
    a = jnp.abs(x)
    e = jnp.exp(-a)
    L = jnp.log(1.0 + e)
    num = jnp.where(x >= 0.0, 1.0, e)
    p = num / (1.0 + e)
    loss_bg = (1.0 - ALPHA) * (jnp.maximum(x, 0.0) + L) * p * p
    s_bg = _dot(loss_bg, jnp.ones((NUM_CLASSES, 1), jnp.float32))
    bg_sum = _dot(inv_mask, s_bg)

    q = 1.0 - lp
    l1 = ALPHA * (lmax0 - ll + lL) * q * q
    l0 = (1.0 - ALPHA) * (lmax0 + lL) * lp * lp
    corr = _dot(ones_g, match * (l1 - l0))
    cls_sum = bg_sum + jnp.sum(inv_mask * corr, axis=1, keepdims=True)

    bt = jax.lax.dot_general(gbox, match, (((0,), (0,)), ((), ())),
                             preferred_element_type=jnp.float32)
    tx1, ty1, tx2, ty2 = (bt[0:1, :], bt[1:2, :], bt[2:3, :], bt[3:4, :])
    a2 = jnp.clip(tx2 - tx1, 0.0) * jnp.clip(ty2 - ty1, 0.0)
    iw = jnp.clip(jnp.minimum(px2, tx2) - jnp.maximum(px1, tx1), 0.0)
    ih = jnp.clip(jnp.minimum(py2, ty2) - jnp.maximum(py1, ty1), 0.0)
    inter = iw * ih
    union = area_p + a2 - inter
    iou = inter / jnp.clip(union, 1e-8)
    cw = jnp.clip(jnp.maximum(px2, tx2) - jnp.minimum(px1, tx1), 0.0)
    ch = jnp.clip(jnp.maximum(py2, ty2) - jnp.minimum(py1, ty1), 0.0)
    carea = cw * ch
    gi = iou - (carea - union) / jnp.clip(carea, 1e-8)
    reg_sum = jnp.sum(fg * (1.0 - gi), axis=1, keepdims=True)

    npos = jnp.sum(fg, axis=1, keepdims=True)

    @pl.when(b == 0)
    def _init():
        acc_ref[...] = jnp.zeros_like(acc_ref)

    acc_ref[0:1, :] += cls_sum
    acc_ref[1:2, :] += reg_sum
    acc_ref[2:3, :] += npos

    @pl.when(b == B - 1)
    def _fin():
        num_fgs = jnp.maximum(acc_ref[2:3, :], 1.0)
        cls_ref[...] = acc_ref[0:1, :] / num_fgs
        reg_ref[...] = acc_ref[1:2, :] / num_fgs


@jax.jit
def kernel(pred_cls, pred_box, anchors, mask, gt_labels, gt_bboxes):
    B, M, C = pred_cls.shape
    G = gt_bboxes.shape[1]

    pbt = jnp.transpose(pred_box, (0, 2, 1))
    anc = jnp.transpose(anchors[:, :2], (1, 0))
    mask_f = mask.astype(jnp.float32).reshape(B, 1, M)
    glab = gt_labels.astype(jnp.int32).reshape(B, G, 1)

    out_sd = jax.ShapeDtypeStruct((1, 1), jnp.float32)
    cls_s, reg_s = pl.pallas_call(
        _ota_kernel,
        grid=(B,),
        in_specs=[
            pl.BlockSpec((1, M, C), lambda b: (b, 0, 0)),
            pl.BlockSpec((1, 4, M), lambda b: (b, 0, 0)),
            pl.BlockSpec((2, M), lambda b: (0, 0)),
            pl.BlockSpec((1, 1, M), lambda b: (b, 0, 0)),
            pl.BlockSpec((1, G, 1), lambda b: (b, 0, 0)),
            pl.BlockSpec((1, G, 4), lambda b: (b, 0, 0)),
        ],
        out_specs=[
            pl.BlockSpec((1, 1), lambda b: (0, 0)),
            pl.BlockSpec((1, 1), lambda b: (0, 0)),
        ],
        out_shape=[out_sd, out_sd],
        scratch_shapes=[pltpu.VMEM((3, 1), jnp.float32)],
    )(pred_cls, pbt, anc, mask_f, glab, gt_bboxes)

    return cls_s[0, 0], reg_s[0, 0]
